# baseline (speedup 1.0000x reference)
import jax
import jax.numpy as jnp
from jax import lax
from jax.experimental import pallas as pl
from jax.experimental.pallas import tpu as pltpu
from jax.experimental.pallas import tpu_sc as plsc

_NUM_CORES = 2
_NUM_SUBCORES = 16
_NUM_WORKERS = _NUM_CORES * _NUM_SUBCORES
_REP = 4


def _make_sc_body(S, per_worker, rep):
    n_dma = per_worker // rep

    def body(table_hbm, out_hbm, buf, sem):
        wid = lax.axis_index("s") * _NUM_CORES + lax.axis_index("c")
        base = wid * per_worker
        for i in range(rep):
            pltpu.sync_copy(table_hbm.at[pl.ds(0, S)], buf.at[i])
        copies = []
        for j in range(n_dma):
            copies.append(
                pltpu.async_copy(
                    buf, out_hbm.at[pl.ds(base + j * rep, rep)], sem
                )
            )
        for c in copies:
            c.wait()

    return body


def _tc_claim_body(sc_ref, table_ref, out_ref):
    del sc_ref
    out_ref[...] = jnp.broadcast_to(
        table_ref[...][None], out_ref.shape
    )


def kernel(x, pos_table):
    N, S = x.shape
    D = pos_table.shape[1]
    per_worker = N // _NUM_WORKERS
    assert per_worker * _NUM_WORKERS == N
    rep = _REP
    while per_worker % rep:
        rep //= 2

    mesh = plsc.VectorSubcoreMesh(core_axis_name="c", subcore_axis_name="s")
    sc_fill = pl.kernel(
        _make_sc_body(S, per_worker, rep),
        out_type=jax.ShapeDtypeStruct((N, S, D), jnp.float32),
        mesh=mesh,
        scratch_types=[
            pltpu.VMEM((rep, S, D), jnp.float32),
            pltpu.SemaphoreType.DMA,
        ],
    )
    filled = sc_fill(pos_table)

    return pl.pallas_call(
        _tc_claim_body,
        grid=(1,),
        in_specs=[
            pl.BlockSpec(memory_space=pltpu.MemorySpace.HBM),
            pl.BlockSpec((S, D), lambda i: (0, 0)),
        ],
        out_specs=pl.BlockSpec((8, S, D), lambda i: (0, 0, 0)),
        out_shape=jax.ShapeDtypeStruct((N, S, D), jnp.float32),
        input_output_aliases={0: 0},
    )(filled, pos_table)
